# Initial kernel scaffold; baseline (speedup 1.0000x reference)
import jax
import jax.numpy as jnp
from jax.experimental import pallas as pl


def kernel(node_features, batch_indices, size, W1, W2):
    raise NotImplementedError("write your pallas kernel here")


# speedup vs baseline: 1.6507x; 1.6507x over previous
import functools

import jax
import jax.numpy as jnp
from jax import lax
from jax.experimental import pallas as pl
from jax.experimental.pallas import tpu as pltpu
from jax.experimental.pallas import tpu_sc as plsc

N = 100000
C = 256
S = 512

NC = 2
NS = 16
L = 16
NW = NC * NS
NV = C // L
SEG_PER_W = S // NW

CH = 128
NB = 544

OCT = N // 8
CCH = 120

_mesh = plsc.VectorSubcoreMesh(core_axis_name="c", subcore_axis_name="s")


@functools.partial(
    pl.kernel,
    out_type=[
        jax.ShapeDtypeStruct((S, C), jnp.float32),
        jax.ShapeDtypeStruct((S, C), jnp.float32),
    ],
    mesh=_mesh,
    scratch_types=[
        pltpu.VMEM((CH, C), jnp.float32),
        pltpu.VMEM((SEG_PER_W, C), jnp.float32),
        pltpu.VMEM((SEG_PER_W, C), jnp.float32),
        pltpu.VMEM((NB,), jnp.int32),
    ],
)
def _seg_reduce(x_hbm, bounds_hbm, sum_hbm, max_hbm, xbuf, sacc, macc, bbuf):
    w = lax.axis_index("c") * NS + lax.axis_index("s")
    base_seg = w * SEG_PER_W

    pltpu.sync_copy(bounds_hbm, bbuf)

    zv = jnp.zeros((L,), jnp.float32)
    nv = jnp.full((L,), -jnp.inf, jnp.float32)

    def seg_body(ls, _):
        bv = bbuf[pl.ds(base_seg + ls, L)]
        s_row = bv[0]
        e_row = bv[1]

        d0 = jnp.minimum(s_row - (s_row & 7), N - CH)
        nch = jnp.where(e_row > s_row,
                        lax.div(e_row - d0 + CH - 1, CH), 0)

        def chunk_body(_, carry):
            p, svs, mvs = carry
            d = pl.multiple_of(jnp.minimum(p - (p & 7), N - CH), 8)
            off = p - d
            c = jnp.minimum(CH - off, e_row - p)
            pltpu.sync_copy(x_hbm.at[pl.ds(d, CH)], xbuf)

            def row_body(jj, rc):
                rsv, rmv = rc
                nsv = []
                nmv = []
                for k in range(NV):
                    xk = xbuf[jj, pl.ds(k * L, L)]
                    nsv.append(rsv[k] + xk)
                    nmv.append(jnp.maximum(rmv[k], xk))
                return (tuple(nsv), tuple(nmv))

            svs, mvs = lax.fori_loop(off, off + c, row_body, (svs, mvs))
            return (p + c, svs, mvs)

        svs0 = tuple(zv for _ in range(NV))
        mvs0 = tuple(nv for _ in range(NV))
        _, svs, mvs = lax.fori_loop(0, nch, chunk_body, (s_row, svs0, mvs0))

        for k in range(NV):
            sl = pl.ds(k * L, L)
            sacc[ls, sl] = svs[k]
            macc[ls, sl] = jnp.where(mvs[k] == -jnp.inf, 0.0, mvs[k])
        return 0

    lax.fori_loop(0, SEG_PER_W, seg_body, 0)

    ob = pl.multiple_of(base_seg, 8)
    pltpu.sync_copy(sacc, sum_hbm.at[pl.ds(ob, SEG_PER_W)])
    pltpu.sync_copy(macc, max_hbm.at[pl.ds(ob, SEG_PER_W)])


def _mlp_body(sum_ref, max_ref, w1_ref, w2_ref, out_ref):
    w1 = w1_ref[...]
    w2 = w2_ref[...]
    dn = (((1,), (1,)), ((), ()))
    hmax = jnp.maximum(
        lax.dot_general(max_ref[...], w1, dn,
                        preferred_element_type=jnp.float32), 0.0)
    hsum = jnp.maximum(
        lax.dot_general(sum_ref[...], w1, dn,
                        preferred_element_type=jnp.float32), 0.0)
    o = (lax.dot_general(hmax, w2, dn, preferred_element_type=jnp.float32)
         + lax.dot_general(hsum, w2, dn, preferred_element_type=jnp.float32))
    out_ref[...] = jax.nn.sigmoid(o)


@functools.partial(
    pl.kernel,
    out_type=jax.ShapeDtypeStruct((N, C), jnp.float32),
    mesh=_mesh,
    scratch_types=[
        pltpu.VMEM((CCH, C), jnp.float32),
        pltpu.VMEM((CCH, C), jnp.float32),
        pltpu.VMEM((CCH,), jnp.int32),
        pltpu.SemaphoreType.DMA,
    ],
)
def _apply_weights(x_hbm, idx_hbm, ref_hbm, out_hbm, xbuf, wbuf, ibuf, sem):
    w = lax.axis_index("c") * NS + lax.axis_index("s")
    p0 = (w * OCT // NW) * 8
    p1 = ((w + 1) * OCT // NW) * 8
    nch = lax.div(p1 - p0 + CCH - 1, CCH)

    def chunk_body(ci, p):
        pp = pl.multiple_of(jnp.minimum(p, p1 - CCH), 8)
        pltpu.sync_copy(x_hbm.at[pl.ds(pp, CCH)], xbuf)
        pltpu.sync_copy(idx_hbm.at[pl.ds(pp, CCH)], ibuf)
        pltpu.async_copy(ref_hbm.at[ibuf], wbuf, sem).wait()

        def row_body(j, _):
            for k in range(NV):
                sl = pl.ds(k * L, L)
                xbuf[j, sl] = xbuf[j, sl] * wbuf[j, sl]
            return 0

        lax.fori_loop(0, CCH, row_body, 0)
        pltpu.sync_copy(xbuf, out_hbm.at[pl.ds(pp, CCH)])
        return p + CCH

    lax.fori_loop(0, nch, chunk_body, p0)


def kernel(node_features, batch_indices, size, W1, W2):
    del size
    idx = batch_indices.astype(jnp.int32)
    edges = jnp.arange(0, S + 1, dtype=jnp.int32)
    bounds = jnp.searchsorted(idx, edges, side="left").astype(jnp.int32)
    bounds = jnp.concatenate([bounds, jnp.full((NB - S - 1,), N, jnp.int32)])

    sum_agg, max_agg = _seg_reduce(node_features, bounds)

    refined = pl.pallas_call(
        _mlp_body,
        out_shape=jax.ShapeDtypeStruct((S, C), jnp.float32),
    )(sum_agg, max_agg, W1, W2)

    return _apply_weights(node_features, idx, refined)


# speedup vs baseline: 3.4135x; 2.0679x over previous
import functools

import jax
import jax.numpy as jnp
from jax import lax
from jax.experimental import pallas as pl
from jax.experimental.pallas import tpu as pltpu
from jax.experimental.pallas import tpu_sc as plsc

N = 100000
C = 256
S = 512

NC = 2
NS = 16
L = 16
NW = NC * NS
NV = C // L
SEG_PER_W = S // NW

CH = 128
NB = 544

OCT = N // 8
CCH = 120

_mesh = plsc.VectorSubcoreMesh(core_axis_name="c", subcore_axis_name="s")


@functools.partial(
    pl.kernel,
    out_type=[
        jax.ShapeDtypeStruct((S, C), jnp.float32),
        jax.ShapeDtypeStruct((S, C), jnp.float32),
    ],
    mesh=_mesh,
    scratch_types=[
        pltpu.VMEM((CH, C), jnp.float32),
        pltpu.VMEM((SEG_PER_W, C), jnp.float32),
        pltpu.VMEM((SEG_PER_W, C), jnp.float32),
        pltpu.VMEM((NB,), jnp.int32),
    ],
)
def _seg_reduce(x_hbm, bounds_hbm, sum_hbm, max_hbm, xbuf, sacc, macc, bbuf):
    w = lax.axis_index("c") * NS + lax.axis_index("s")
    base_seg = w * SEG_PER_W

    pltpu.sync_copy(bounds_hbm, bbuf)

    zv = jnp.zeros((L,), jnp.float32)
    nv = jnp.full((L,), -jnp.inf, jnp.float32)

    def seg_body(ls, _):
        bv = bbuf[pl.ds(base_seg + ls, L)]
        s_row = bv[0]
        e_row = bv[1]

        d0 = jnp.minimum(s_row - (s_row & 7), N - CH)
        nch = jnp.where(e_row > s_row,
                        lax.div(e_row - d0 + CH - 1, CH), 0)

        def chunk_body(_, carry):
            p, svs, mvs = carry
            d = pl.multiple_of(jnp.minimum(p - (p & 7), N - CH), 8)
            off = p - d
            c = jnp.minimum(CH - off, e_row - p)
            pltpu.sync_copy(x_hbm.at[pl.ds(d, CH)], xbuf)

            def row_body(jj, rc):
                rsv, rmv = rc
                nsv = []
                nmv = []
                for k in range(NV):
                    xk = xbuf[jj, pl.ds(k * L, L)]
                    nsv.append(rsv[k] + xk)
                    nmv.append(jnp.maximum(rmv[k], xk))
                return (tuple(nsv), tuple(nmv))

            svs, mvs = lax.fori_loop(off, off + c, row_body, (svs, mvs))
            return (p + c, svs, mvs)

        svs0 = tuple(zv for _ in range(NV))
        mvs0 = tuple(nv for _ in range(NV))
        _, svs, mvs = lax.fori_loop(0, nch, chunk_body, (s_row, svs0, mvs0))

        for k in range(NV):
            sl = pl.ds(k * L, L)
            sacc[ls, sl] = svs[k]
            macc[ls, sl] = jnp.where(mvs[k] == -jnp.inf, 0.0, mvs[k])
        return 0

    lax.fori_loop(0, SEG_PER_W, seg_body, 0)

    ob = pl.multiple_of(base_seg, 8)
    pltpu.sync_copy(sacc, sum_hbm.at[pl.ds(ob, SEG_PER_W)])
    pltpu.sync_copy(macc, max_hbm.at[pl.ds(ob, SEG_PER_W)])


def _mlp_body(sum_ref, max_ref, w1_ref, w2_ref, out_ref):
    w1 = w1_ref[...]
    w2 = w2_ref[...]
    dn = (((1,), (1,)), ((), ()))
    hmax = jnp.maximum(
        lax.dot_general(max_ref[...], w1, dn,
                        preferred_element_type=jnp.float32), 0.0)
    hsum = jnp.maximum(
        lax.dot_general(sum_ref[...], w1, dn,
                        preferred_element_type=jnp.float32), 0.0)
    o = (lax.dot_general(hmax, w2, dn, preferred_element_type=jnp.float32)
         + lax.dot_general(hsum, w2, dn, preferred_element_type=jnp.float32))
    out_ref[...] = jax.nn.sigmoid(o)


@functools.partial(
    pl.kernel,
    out_type=jax.ShapeDtypeStruct((N, C), jnp.float32),
    mesh=_mesh,
    scratch_types=[
        pltpu.VMEM((CCH, C), jnp.float32),
        pltpu.VMEM((CCH, C), jnp.float32),
        pltpu.VMEM((8, C), jnp.float32),
        pltpu.VMEM((NB,), jnp.int32),
    ],
)
def _apply_weights(x_hbm, bounds_hbm, ref_hbm, out_hbm, xbuf, obuf, wrow,
                   bbuf):
    w = lax.axis_index("c") * NS + lax.axis_index("s")
    p0 = (w * OCT // NW) * 8
    p1 = ((w + 1) * OCT // NW) * 8
    nch = lax.div(p1 - p0 + CCH - 1, CCH)

    pltpu.sync_copy(bounds_hbm, bbuf)

    def br(v):
        def step(_, st):
            lo, hi = st
            mid = lax.div(lo + hi, 2)
            big = bbuf[pl.ds(mid, L)][0] > v
            return (jnp.where(big, lo, mid), jnp.where(big, mid, hi))

        lo, _ = lax.fori_loop(0, 10, step, (0, 513))
        return lo

    def chunk_body(ci, p):
        pp = pl.multiple_of(jnp.minimum(p, p1 - CCH), 8)
        pltpu.sync_copy(x_hbm.at[pl.ds(pp, CCH)], xbuf)

        s_lo = br(pp)
        s_hi = br(pp + CCH - 1)

        def run_body(ri, _):
            s = s_lo + ri
            bv = bbuf[pl.ds(s, L)]
            a = jnp.maximum(bv[0], pp) - pp
            e = jnp.minimum(bv[1], pp + CCH) - pp
            sa = pl.multiple_of(s - (s & 7), 8)
            pltpu.sync_copy(ref_hbm.at[pl.ds(sa, 8)], wrow)
            wv = [wrow[s & 7, pl.ds(k * L, L)] for k in range(NV)]

            def rowm(j, _):
                for k in range(NV):
                    sl = pl.ds(k * L, L)
                    obuf[j, sl] = xbuf[j, sl] * wv[k]
                return 0

            lax.fori_loop(a, e, rowm, 0)
            return 0

        lax.fori_loop(0, s_hi - s_lo + 1, run_body, 0)
        pltpu.sync_copy(obuf, out_hbm.at[pl.ds(pp, CCH)])
        return p + CCH

    lax.fori_loop(0, nch, chunk_body, p0)


def kernel(node_features, batch_indices, size, W1, W2):
    del size
    idx = batch_indices.astype(jnp.int32)
    edges = jnp.arange(0, S + 1, dtype=jnp.int32)
    bounds = jnp.searchsorted(idx, edges, side="left").astype(jnp.int32)
    bounds = jnp.concatenate([bounds, jnp.full((NB - S - 1,), N, jnp.int32)])

    sum_agg, max_agg = _seg_reduce(node_features, bounds)

    refined = pl.pallas_call(
        _mlp_body,
        out_shape=jax.ShapeDtypeStruct((S, C), jnp.float32),
    )(sum_agg, max_agg, W1, W2)

    return _apply_weights(node_features, bounds, refined)


# speedup vs baseline: 3.8226x; 1.1198x over previous
import functools

import jax
import jax.numpy as jnp
from jax import lax
from jax.experimental import pallas as pl
from jax.experimental.pallas import tpu as pltpu
from jax.experimental.pallas import tpu_sc as plsc

N = 100000
C = 256
S = 512

NC = 2
NS = 16
L = 16
NW = NC * NS
NV = C // L
SEG_PER_W = S // NW

CH = 128
NB = 544

OCT = N // 8
CCH = 120

_mesh = plsc.VectorSubcoreMesh(core_axis_name="c", subcore_axis_name="s")


@functools.partial(
    pl.kernel,
    out_type=[
        jax.ShapeDtypeStruct((S, C), jnp.float32),
        jax.ShapeDtypeStruct((S, C), jnp.float32),
    ],
    mesh=_mesh,
    scratch_types=[
        pltpu.VMEM((CH, C), jnp.float32),
        pltpu.VMEM((CH, C), jnp.float32),
        pltpu.VMEM((SEG_PER_W + 1, C), jnp.float32),
        pltpu.VMEM((SEG_PER_W + 1, C), jnp.float32),
        pltpu.VMEM((NB,), jnp.int32),
        pltpu.SemaphoreType.DMA,
        pltpu.SemaphoreType.DMA,
    ],
)
def _seg_reduce(x_hbm, bounds_hbm, sum_hbm, max_hbm, xbuf0, xbuf1,
                sacc, macc, bbuf, sem0, sem1):
    w = lax.axis_index("c") * NS + lax.axis_index("s")
    base_seg = w * SEG_PER_W

    pltpu.sync_copy(bounds_hbm, bbuf)
    r0 = bbuf[pl.ds(base_seg, L)][0]
    r1 = bbuf[pl.ds(base_seg + SEG_PER_W, L)][0]
    d0 = pl.multiple_of(r0 - (r0 & 7), 8)
    nch = jnp.where(r1 > r0, lax.div(r1 - d0 + CH - 1, CH), 0)

    zv = jnp.zeros((L,), jnp.float32)
    nv = jnp.full((L,), -jnp.inf, jnp.float32)

    def init_body(i, _):
        for k in range(NV):
            sl = pl.ds(k * L, L)
            sacc[i, sl] = zv
            macc[i, sl] = nv
        return 0

    lax.fori_loop(0, SEG_PER_W, init_body, 0)

    def chunk_base(k):
        return pl.multiple_of(
            jnp.minimum(jnp.maximum(d0 + k * CH, 0), N - CH), 8)

    def dma(k, buf, sem):
        return pltpu.make_async_copy(
            x_hbm.at[pl.ds(chunk_base(k), CH)], buf, sem)

    def process(k, buf, carry):
        d = chunk_base(k)
        lo = jnp.maximum(r0, d0 + k * CH)
        hi = jnp.minimum(r1, d0 + (k + 1) * CH)

        def seg_scan(ls, sc):
            svs, mvs = sc
            bv = bbuf[pl.ds(base_seg + ls, L)]
            s_row = bv[0]
            e_row = bv[1]
            a = jnp.maximum(s_row, lo)
            b = jnp.maximum(jnp.minimum(e_row, hi), a)

            def row_body(jj, rc):
                rsv, rmv = rc
                nsv = []
                nmv = []
                for k16 in range(NV):
                    xk = buf[jj, pl.ds(k16 * L, L)]
                    nsv.append(rsv[k16] + xk)
                    nmv.append(jnp.maximum(rmv[k16], xk))
                return (tuple(nsv), tuple(nmv))

            svs, mvs = lax.fori_loop(a - d, b - d, row_body, (svs, mvs))

            inter = b > a
            tgt = jnp.where(inter, ls, SEG_PER_W)
            for k16 in range(NV):
                sl = pl.ds(k16 * L, L)
                sacc[tgt, sl] = svs[k16]
                macc[tgt, sl] = mvs[k16]
            ended = inter & (e_row <= hi)
            svs = tuple(jnp.where(ended, zv, v) for v in svs)
            mvs = tuple(jnp.where(ended, nv, v) for v in mvs)
            return (svs, mvs)

        return lax.fori_loop(0, SEG_PER_W, seg_scan, carry)

    carry = (tuple(zv for _ in range(NV)), tuple(nv for _ in range(NV)))
    dma(0, xbuf0, sem0).start()
    npairs = lax.div(nch + 1, 2)

    def pair_body(i, carry):
        dma(2 * i + 1, xbuf1, sem1).start()
        dma(2 * i, xbuf0, sem0).wait()
        carry = process(2 * i, xbuf0, carry)
        dma(2 * i + 2, xbuf0, sem0).start()
        dma(2 * i + 1, xbuf1, sem1).wait()
        carry = process(2 * i + 1, xbuf1, carry)
        return carry

    carry = lax.fori_loop(0, npairs, pair_body, carry)
    dma(2 * npairs, xbuf0, sem0).wait()

    def fin_body(i, _):
        for k in range(NV):
            sl = pl.ds(k * L, L)
            v = macc[i, sl]
            macc[i, sl] = jnp.where(v == -jnp.inf, 0.0, v)
        return 0

    lax.fori_loop(0, SEG_PER_W, fin_body, 0)

    ob = pl.multiple_of(base_seg, 8)
    pltpu.sync_copy(sacc.at[pl.ds(0, SEG_PER_W)],
                    sum_hbm.at[pl.ds(ob, SEG_PER_W)])
    pltpu.sync_copy(macc.at[pl.ds(0, SEG_PER_W)],
                    max_hbm.at[pl.ds(ob, SEG_PER_W)])


def _mlp_body(sum_ref, max_ref, w1_ref, w2_ref, out_ref):
    w1 = w1_ref[...]
    w2 = w2_ref[...]
    dn = (((1,), (1,)), ((), ()))
    hmax = jnp.maximum(
        lax.dot_general(max_ref[...], w1, dn,
                        preferred_element_type=jnp.float32), 0.0)
    hsum = jnp.maximum(
        lax.dot_general(sum_ref[...], w1, dn,
                        preferred_element_type=jnp.float32), 0.0)
    o = (lax.dot_general(hmax, w2, dn, preferred_element_type=jnp.float32)
         + lax.dot_general(hsum, w2, dn, preferred_element_type=jnp.float32))
    out_ref[...] = jax.nn.sigmoid(o)


@functools.partial(
    pl.kernel,
    out_type=jax.ShapeDtypeStruct((N, C), jnp.float32),
    mesh=_mesh,
    scratch_types=[
        pltpu.VMEM((CCH, C), jnp.float32),
        pltpu.VMEM((CCH, C), jnp.float32),
        pltpu.VMEM((8, C), jnp.float32),
        pltpu.VMEM((NB,), jnp.int32),
    ],
)
def _apply_weights(x_hbm, bounds_hbm, ref_hbm, out_hbm, xbuf, obuf, wrow,
                   bbuf):
    w = lax.axis_index("c") * NS + lax.axis_index("s")
    p0 = (w * OCT // NW) * 8
    p1 = ((w + 1) * OCT // NW) * 8
    nch = lax.div(p1 - p0 + CCH - 1, CCH)

    pltpu.sync_copy(bounds_hbm, bbuf)

    def br(v):
        def step(_, st):
            lo, hi = st
            mid = lax.div(lo + hi, 2)
            big = bbuf[pl.ds(mid, L)][0] > v
            return (jnp.where(big, lo, mid), jnp.where(big, mid, hi))

        lo, _ = lax.fori_loop(0, 10, step, (0, 513))
        return lo

    def chunk_body(ci, p):
        pp = pl.multiple_of(jnp.minimum(p, p1 - CCH), 8)
        pltpu.sync_copy(x_hbm.at[pl.ds(pp, CCH)], xbuf)

        s_lo = br(pp)
        s_hi = br(pp + CCH - 1)

        def run_body(ri, _):
            s = s_lo + ri
            bv = bbuf[pl.ds(s, L)]
            a = jnp.maximum(bv[0], pp) - pp
            e = jnp.minimum(bv[1], pp + CCH) - pp
            sa = pl.multiple_of(s - (s & 7), 8)
            pltpu.sync_copy(ref_hbm.at[pl.ds(sa, 8)], wrow)
            wv = [wrow[s & 7, pl.ds(k * L, L)] for k in range(NV)]

            def rowm(j, _):
                for k in range(NV):
                    sl = pl.ds(k * L, L)
                    obuf[j, sl] = xbuf[j, sl] * wv[k]
                return 0

            lax.fori_loop(a, e, rowm, 0)
            return 0

        lax.fori_loop(0, s_hi - s_lo + 1, run_body, 0)
        pltpu.sync_copy(obuf, out_hbm.at[pl.ds(pp, CCH)])
        return p + CCH

    lax.fori_loop(0, nch, chunk_body, p0)


def kernel(node_features, batch_indices, size, W1, W2):
    del size
    idx = batch_indices.astype(jnp.int32)
    edges = jnp.arange(0, S + 1, dtype=jnp.int32)
    bounds = jnp.searchsorted(idx, edges, side="left").astype(jnp.int32)
    bounds = jnp.concatenate([bounds, jnp.full((NB - S - 1,), N, jnp.int32)])

    sum_agg, max_agg = _seg_reduce(node_features, bounds)

    refined = pl.pallas_call(
        _mlp_body,
        out_shape=jax.ShapeDtypeStruct((S, C), jnp.float32),
    )(sum_agg, max_agg, W1, W2)

    return _apply_weights(node_features, bounds, refined)


# speedup vs baseline: 4.6546x; 1.2177x over previous
import functools

import jax
import jax.numpy as jnp
from jax import lax
from jax.experimental import pallas as pl
from jax.experimental.pallas import tpu as pltpu
from jax.experimental.pallas import tpu_sc as plsc

N = 100000
C = 256
S = 512

NC = 2
NS = 16
L = 16
NW = NC * NS
NV = C // L
SEG_PER_W = S // NW

CH = 128
NB = 544

OCT = N // 8
CCH = 120

_mesh = plsc.VectorSubcoreMesh(core_axis_name="c", subcore_axis_name="s")


@functools.partial(
    pl.kernel,
    out_type=[
        jax.ShapeDtypeStruct((S, C), jnp.float32),
        jax.ShapeDtypeStruct((S, C), jnp.float32),
    ],
    mesh=_mesh,
    scratch_types=[
        pltpu.VMEM((CH, C), jnp.float32),
        pltpu.VMEM((CH, C), jnp.float32),
        pltpu.VMEM((SEG_PER_W + 1, C), jnp.float32),
        pltpu.VMEM((SEG_PER_W + 1, C), jnp.float32),
        pltpu.VMEM((NB,), jnp.int32),
        pltpu.SemaphoreType.DMA,
        pltpu.SemaphoreType.DMA,
    ],
)
def _seg_reduce(x_hbm, bounds_hbm, sum_hbm, max_hbm, xbuf0, xbuf1,
                sacc, macc, bbuf, sem0, sem1):
    w = lax.axis_index("c") * NS + lax.axis_index("s")
    base_seg = w * SEG_PER_W

    pltpu.sync_copy(bounds_hbm, bbuf)
    r0 = bbuf[pl.ds(base_seg, L)][0]
    r1 = bbuf[pl.ds(base_seg + SEG_PER_W, L)][0]
    d0 = pl.multiple_of(r0 - (r0 & 7), 8)
    nch = jnp.where(r1 > r0, lax.div(r1 - d0 + CH - 1, CH), 0)

    zv = jnp.zeros((L,), jnp.float32)
    nv = jnp.full((L,), -jnp.inf, jnp.float32)

    def init_body(i, _):
        for k in range(NV):
            sl = pl.ds(k * L, L)
            sacc[i, sl] = zv
            macc[i, sl] = nv
        return 0

    lax.fori_loop(0, SEG_PER_W, init_body, 0)

    def chunk_base(k):
        return pl.multiple_of(
            jnp.minimum(jnp.maximum(d0 + k * CH, 0), N - CH), 8)

    def dma(k, buf, sem):
        return pltpu.make_async_copy(
            x_hbm.at[pl.ds(chunk_base(k), CH)], buf, sem)

    def process(k, buf, carry):
        d = chunk_base(k)
        lo = jnp.maximum(r0, d0 + k * CH)
        hi = jnp.minimum(r1, d0 + (k + 1) * CH)

        def seg_scan(ls, sc):
            svs, mvs = sc
            bv = bbuf[pl.ds(base_seg + ls, L)]
            s_row = bv[0]
            e_row = bv[1]
            a = jnp.maximum(s_row, lo)
            b = jnp.maximum(jnp.minimum(e_row, hi), a)

            def row_body(jj, rc):
                rsv, rmv = rc
                nsv = []
                nmv = []
                for k16 in range(NV):
                    xk = buf[jj, pl.ds(k16 * L, L)]
                    nsv.append(rsv[k16] + xk)
                    nmv.append(jnp.maximum(rmv[k16], xk))
                return (tuple(nsv), tuple(nmv))

            svs, mvs = lax.fori_loop(a - d, b - d, row_body, (svs, mvs))

            inter = b > a
            tgt = jnp.where(inter, ls, SEG_PER_W)
            for k16 in range(NV):
                sl = pl.ds(k16 * L, L)
                sacc[tgt, sl] = svs[k16]
                macc[tgt, sl] = mvs[k16]
            ended = inter & (e_row <= hi)
            svs = tuple(jnp.where(ended, zv, v) for v in svs)
            mvs = tuple(jnp.where(ended, nv, v) for v in mvs)
            return (svs, mvs)

        return lax.fori_loop(0, SEG_PER_W, seg_scan, carry)

    carry = (tuple(zv for _ in range(NV)), tuple(nv for _ in range(NV)))
    dma(0, xbuf0, sem0).start()
    npairs = lax.div(nch + 1, 2)

    def pair_body(i, carry):
        dma(2 * i + 1, xbuf1, sem1).start()
        dma(2 * i, xbuf0, sem0).wait()
        carry = process(2 * i, xbuf0, carry)
        dma(2 * i + 2, xbuf0, sem0).start()
        dma(2 * i + 1, xbuf1, sem1).wait()
        carry = process(2 * i + 1, xbuf1, carry)
        return carry

    carry = lax.fori_loop(0, npairs, pair_body, carry)
    dma(2 * npairs, xbuf0, sem0).wait()

    def fin_body(i, _):
        for k in range(NV):
            sl = pl.ds(k * L, L)
            v = macc[i, sl]
            macc[i, sl] = jnp.where(v == -jnp.inf, 0.0, v)
        return 0

    lax.fori_loop(0, SEG_PER_W, fin_body, 0)

    ob = pl.multiple_of(base_seg, 8)
    pltpu.sync_copy(sacc.at[pl.ds(0, SEG_PER_W)],
                    sum_hbm.at[pl.ds(ob, SEG_PER_W)])
    pltpu.sync_copy(macc.at[pl.ds(0, SEG_PER_W)],
                    max_hbm.at[pl.ds(ob, SEG_PER_W)])


def _mlp_body(sum_ref, max_ref, w1_ref, w2_ref, out_ref):
    w1 = w1_ref[...]
    w2 = w2_ref[...]
    dn = (((1,), (1,)), ((), ()))
    hmax = jnp.maximum(
        lax.dot_general(max_ref[...], w1, dn,
                        preferred_element_type=jnp.float32), 0.0)
    hsum = jnp.maximum(
        lax.dot_general(sum_ref[...], w1, dn,
                        preferred_element_type=jnp.float32), 0.0)
    o = (lax.dot_general(hmax, w2, dn, preferred_element_type=jnp.float32)
         + lax.dot_general(hsum, w2, dn, preferred_element_type=jnp.float32))
    out_ref[...] = jax.nn.sigmoid(o)


@functools.partial(
    pl.kernel,
    out_type=jax.ShapeDtypeStruct((N, C), jnp.float32),
    mesh=_mesh,
    scratch_types=[
        pltpu.VMEM((CCH, C), jnp.float32),
        pltpu.VMEM((CCH, C), jnp.float32),
        pltpu.VMEM((CCH, C), jnp.float32),
        pltpu.VMEM((CCH, C), jnp.float32),
        pltpu.VMEM((8, C), jnp.float32),
        pltpu.VMEM((NB,), jnp.int32),
        pltpu.SemaphoreType.DMA,
        pltpu.SemaphoreType.DMA,
        pltpu.SemaphoreType.DMA,
        pltpu.SemaphoreType.DMA,
    ],
)
def _apply_weights(x_hbm, bounds_hbm, ref_hbm, out_hbm, xin0, xin1,
                   ob0, ob1, wrow, bbuf, sx0, sx1, so0, so1):
    w = lax.axis_index("c") * NS + lax.axis_index("s")
    p0 = (w * OCT // NW) * 8
    p1 = ((w + 1) * OCT // NW) * 8
    nch = lax.div(p1 - p0 + CCH - 1, CCH)

    pltpu.sync_copy(bounds_hbm, bbuf)

    def chunk_pp(k):
        return pl.multiple_of(jnp.minimum(p0 + k * CCH, p1 - CCH), 8)

    def in_dma(k, buf, sem):
        return pltpu.make_async_copy(
            x_hbm.at[pl.ds(chunk_pp(k), CCH)], buf, sem)

    def out_dma(k, buf, sem):
        return pltpu.make_async_copy(
            buf, out_hbm.at[pl.ds(chunk_pp(k), CCH)], sem)

    def br(v):
        def step(_, st):
            lo, hi = st
            mid = lax.div(lo + hi, 2)
            big = bbuf[pl.ds(mid, L)][0] > v
            return (jnp.where(big, lo, mid), jnp.where(big, mid, hi))

        lo, _ = lax.fori_loop(0, 10, step, (0, 513))
        return lo

    def process(k, xin, obuf):
        pp = chunk_pp(k)
        s_lo = br(pp)
        s_hi = br(pp + CCH - 1)

        def run_body(ri, _):
            s = s_lo + ri
            bv = bbuf[pl.ds(s, L)]
            a = jnp.maximum(bv[0], pp) - pp
            e = jnp.minimum(bv[1], pp + CCH) - pp
            sa = pl.multiple_of(s - (s & 7), 8)
            pltpu.sync_copy(ref_hbm.at[pl.ds(sa, 8)], wrow)
            wv = [wrow[s & 7, pl.ds(k16 * L, L)] for k16 in range(NV)]

            def rowm(j, _):
                for k16 in range(NV):
                    sl = pl.ds(k16 * L, L)
                    obuf[j, sl] = xin[j, sl] * wv[k16]
                return 0

            lax.fori_loop(a, e, rowm, 0)
            return 0

        lax.fori_loop(0, s_hi - s_lo + 1, run_body, 0)

    in_dma(0, xin0, sx0).start()
    in_dma(1, xin1, sx1).start()

    in_dma(0, xin0, sx0).wait()
    process(0, xin0, ob0)
    out_dma(0, ob0, so0).start()
    in_dma(2, xin0, sx0).start()

    in_dma(1, xin1, sx1).wait()
    process(1, xin1, ob1)
    out_dma(1, ob1, so1).start()
    in_dma(3, xin1, sx1).start()

    npr = lax.div(nch - 1, 2)

    def pair_body(i, _):
        k0 = 2 * i + 2
        in_dma(k0, xin0, sx0).wait()
        out_dma(k0 - 2, ob0, so0).wait()
        process(k0, xin0, ob0)
        out_dma(k0, ob0, so0).start()
        in_dma(k0 + 2, xin0, sx0).start()

        k1 = k0 + 1
        in_dma(k1, xin1, sx1).wait()
        out_dma(k1 - 2, ob1, so1).wait()
        process(k1, xin1, ob1)
        out_dma(k1, ob1, so1).start()
        in_dma(k1 + 2, xin1, sx1).start()
        return 0

    lax.fori_loop(0, npr, pair_body, 0)

    K = 2 * npr + 1
    out_dma(K - 1, ob0, so0).wait()
    out_dma(K, ob1, so1).wait()
    in_dma(K + 1, xin0, sx0).wait()
    in_dma(K + 2, xin1, sx1).wait()


def kernel(node_features, batch_indices, size, W1, W2):
    del size
    idx = batch_indices.astype(jnp.int32)
    edges = jnp.arange(0, S + 1, dtype=jnp.int32)
    bounds = jnp.searchsorted(idx, edges, side="left").astype(jnp.int32)
    bounds = jnp.concatenate([bounds, jnp.full((NB - S - 1,), N, jnp.int32)])

    sum_agg, max_agg = _seg_reduce(node_features, bounds)

    refined = pl.pallas_call(
        _mlp_body,
        out_shape=jax.ShapeDtypeStruct((S, C), jnp.float32),
    )(sum_agg, max_agg, W1, W2)

    return _apply_weights(node_features, bounds, refined)


# speedup vs baseline: 4.7520x; 1.0209x over previous
import functools

import jax
import jax.numpy as jnp
from jax import lax
from jax.experimental import pallas as pl
from jax.experimental.pallas import tpu as pltpu
from jax.experimental.pallas import tpu_sc as plsc

N = 100000
C = 256
S = 512

NC = 2
NS = 16
L = 16
NW = NC * NS
NV = C // L
SEG_PER_W = S // NW

CH = 128
NB = 544

OCT = N // 8
CCH = 120

_mesh = plsc.VectorSubcoreMesh(core_axis_name="c", subcore_axis_name="s")


@functools.partial(
    pl.kernel,
    out_type=[
        jax.ShapeDtypeStruct((S, C), jnp.float32),
        jax.ShapeDtypeStruct((S, C), jnp.float32),
    ],
    mesh=_mesh,
    scratch_types=[
        pltpu.VMEM((CH, C), jnp.float32),
        pltpu.VMEM((CH, C), jnp.float32),
        pltpu.VMEM((SEG_PER_W + 1, C), jnp.float32),
        pltpu.VMEM((SEG_PER_W + 1, C), jnp.float32),
        pltpu.VMEM((NB,), jnp.int32),
        pltpu.SemaphoreType.DMA,
        pltpu.SemaphoreType.DMA,
    ],
)
def _seg_reduce(x_hbm, bounds_hbm, sum_hbm, max_hbm, xbuf0, xbuf1,
                sacc, macc, bbuf, sem0, sem1):
    w = lax.axis_index("c") * NS + lax.axis_index("s")
    base_seg = w * SEG_PER_W

    pltpu.sync_copy(bounds_hbm, bbuf)
    r0 = bbuf[pl.ds(base_seg, L)][0]
    r1 = bbuf[pl.ds(base_seg + SEG_PER_W, L)][0]
    d0 = pl.multiple_of(r0 - (r0 & 7), 8)
    nch = jnp.where(r1 > r0, lax.div(r1 - d0 + CH - 1, CH), 0)

    zv = jnp.zeros((L,), jnp.float32)
    nv = jnp.full((L,), -jnp.inf, jnp.float32)

    def init_body(i, _):
        for k in range(NV):
            sl = pl.ds(k * L, L)
            sacc[i, sl] = zv
            macc[i, sl] = nv
        return 0

    lax.fori_loop(0, SEG_PER_W, init_body, 0)

    def chunk_base(k):
        return pl.multiple_of(
            jnp.minimum(jnp.maximum(d0 + k * CH, 0), N - CH), 8)

    def dma(k, buf, sem):
        return pltpu.make_async_copy(
            x_hbm.at[pl.ds(chunk_base(k), CH)], buf, sem)

    def wbr(v):
        def step(_, st):
            blo, bhi = st
            mid = lax.div(blo + bhi, 2)
            big = bbuf[pl.ds(mid, L)][0] > v
            return (jnp.where(big, blo, mid), jnp.where(big, mid, bhi))

        blo, _ = lax.fori_loop(0, 5, step,
                               (base_seg, base_seg + SEG_PER_W + 1))
        return blo

    def process(k, buf, carry):
        d = chunk_base(k)
        lo = jnp.maximum(r0, d0 + k * CH)
        hi = jnp.minimum(r1, d0 + (k + 1) * CH)
        ls_lo = wbr(lo) - base_seg
        ls_hi = wbr(hi - 1) - base_seg

        def seg_scan(ls, sc):
            svs, mvs = sc
            bv = bbuf[pl.ds(base_seg + ls, L)]
            s_row = bv[0]
            e_row = bv[1]
            a = jnp.maximum(s_row, lo)
            b = jnp.maximum(jnp.minimum(e_row, hi), a)

            def row_body(jj, rc):
                rsv, rmv = rc
                nsv = []
                nmv = []
                for k16 in range(NV):
                    xk = buf[jj, pl.ds(k16 * L, L)]
                    nsv.append(rsv[k16] + xk)
                    nmv.append(jnp.maximum(rmv[k16], xk))
                return (tuple(nsv), tuple(nmv))

            svs, mvs = lax.fori_loop(a - d, b - d, row_body, (svs, mvs))

            inter = b > a
            tgt = jnp.where(inter, ls, SEG_PER_W)
            for k16 in range(NV):
                sl = pl.ds(k16 * L, L)
                sacc[tgt, sl] = svs[k16]
                macc[tgt, sl] = mvs[k16]
            ended = inter & (e_row <= hi)
            svs = tuple(jnp.where(ended, zv, v) for v in svs)
            mvs = tuple(jnp.where(ended, nv, v) for v in mvs)
            return (svs, mvs)

        return lax.fori_loop(ls_lo, ls_hi + 1, seg_scan, carry)

    carry = (tuple(zv for _ in range(NV)), tuple(nv for _ in range(NV)))
    dma(0, xbuf0, sem0).start()
    npairs = lax.div(nch + 1, 2)

    def pair_body(i, carry):
        dma(2 * i + 1, xbuf1, sem1).start()
        dma(2 * i, xbuf0, sem0).wait()
        carry = process(2 * i, xbuf0, carry)
        dma(2 * i + 2, xbuf0, sem0).start()
        dma(2 * i + 1, xbuf1, sem1).wait()
        carry = process(2 * i + 1, xbuf1, carry)
        return carry

    carry = lax.fori_loop(0, npairs, pair_body, carry)
    dma(2 * npairs, xbuf0, sem0).wait()

    def fin_body(i, _):
        for k in range(NV):
            sl = pl.ds(k * L, L)
            v = macc[i, sl]
            macc[i, sl] = jnp.where(v == -jnp.inf, 0.0, v)
        return 0

    lax.fori_loop(0, SEG_PER_W, fin_body, 0)

    ob = pl.multiple_of(base_seg, 8)
    pltpu.sync_copy(sacc.at[pl.ds(0, SEG_PER_W)],
                    sum_hbm.at[pl.ds(ob, SEG_PER_W)])
    pltpu.sync_copy(macc.at[pl.ds(0, SEG_PER_W)],
                    max_hbm.at[pl.ds(ob, SEG_PER_W)])


def _mlp_body(sum_ref, max_ref, w1_ref, w2_ref, out_ref):
    w1 = w1_ref[...]
    w2 = w2_ref[...]
    dn = (((1,), (1,)), ((), ()))
    hmax = jnp.maximum(
        lax.dot_general(max_ref[...], w1, dn,
                        preferred_element_type=jnp.float32), 0.0)
    hsum = jnp.maximum(
        lax.dot_general(sum_ref[...], w1, dn,
                        preferred_element_type=jnp.float32), 0.0)
    o = (lax.dot_general(hmax, w2, dn, preferred_element_type=jnp.float32)
         + lax.dot_general(hsum, w2, dn, preferred_element_type=jnp.float32))
    out_ref[...] = jax.nn.sigmoid(o)


@functools.partial(
    pl.kernel,
    out_type=jax.ShapeDtypeStruct((N, C), jnp.float32),
    mesh=_mesh,
    scratch_types=[
        pltpu.VMEM((CCH, C), jnp.float32),
        pltpu.VMEM((CCH, C), jnp.float32),
        pltpu.VMEM((CCH, C), jnp.float32),
        pltpu.VMEM((CCH, C), jnp.float32),
        pltpu.VMEM((8, C), jnp.float32),
        pltpu.VMEM((NB,), jnp.int32),
        pltpu.SemaphoreType.DMA,
        pltpu.SemaphoreType.DMA,
        pltpu.SemaphoreType.DMA,
        pltpu.SemaphoreType.DMA,
    ],
)
def _apply_weights(x_hbm, bounds_hbm, ref_hbm, out_hbm, xin0, xin1,
                   ob0, ob1, wrow, bbuf, sx0, sx1, so0, so1):
    w = lax.axis_index("c") * NS + lax.axis_index("s")
    p0 = (w * OCT // NW) * 8
    p1 = ((w + 1) * OCT // NW) * 8
    nch = lax.div(p1 - p0 + CCH - 1, CCH)

    pltpu.sync_copy(bounds_hbm, bbuf)

    def chunk_pp(k):
        return pl.multiple_of(jnp.minimum(p0 + k * CCH, p1 - CCH), 8)

    def in_dma(k, buf, sem):
        return pltpu.make_async_copy(
            x_hbm.at[pl.ds(chunk_pp(k), CCH)], buf, sem)

    def out_dma(k, buf, sem):
        return pltpu.make_async_copy(
            buf, out_hbm.at[pl.ds(chunk_pp(k), CCH)], sem)

    def br(v):
        def step(_, st):
            lo, hi = st
            mid = lax.div(lo + hi, 2)
            big = bbuf[pl.ds(mid, L)][0] > v
            return (jnp.where(big, lo, mid), jnp.where(big, mid, hi))

        lo, _ = lax.fori_loop(0, 10, step, (0, 513))
        return lo

    def process(k, xin, obuf):
        pp = chunk_pp(k)
        s_lo = br(pp)
        s_hi = br(pp + CCH - 1)

        def run_body(ri, _):
            s = s_lo + ri
            bv = bbuf[pl.ds(s, L)]
            a = jnp.maximum(bv[0], pp) - pp
            e = jnp.minimum(bv[1], pp + CCH) - pp
            sa = pl.multiple_of(s - (s & 7), 8)
            pltpu.sync_copy(ref_hbm.at[pl.ds(sa, 8)], wrow)
            wv = [wrow[s & 7, pl.ds(k16 * L, L)] for k16 in range(NV)]

            def rowm(j, _):
                for k16 in range(NV):
                    sl = pl.ds(k16 * L, L)
                    obuf[j, sl] = xin[j, sl] * wv[k16]
                return 0

            lax.fori_loop(a, e, rowm, 0)
            return 0

        lax.fori_loop(0, s_hi - s_lo + 1, run_body, 0)

    in_dma(0, xin0, sx0).start()
    in_dma(1, xin1, sx1).start()

    in_dma(0, xin0, sx0).wait()
    process(0, xin0, ob0)
    out_dma(0, ob0, so0).start()
    in_dma(2, xin0, sx0).start()

    in_dma(1, xin1, sx1).wait()
    process(1, xin1, ob1)
    out_dma(1, ob1, so1).start()
    in_dma(3, xin1, sx1).start()

    npr = lax.div(nch - 1, 2)

    def pair_body(i, _):
        k0 = 2 * i + 2
        in_dma(k0, xin0, sx0).wait()
        out_dma(k0 - 2, ob0, so0).wait()
        process(k0, xin0, ob0)
        out_dma(k0, ob0, so0).start()
        in_dma(k0 + 2, xin0, sx0).start()

        k1 = k0 + 1
        in_dma(k1, xin1, sx1).wait()
        out_dma(k1 - 2, ob1, so1).wait()
        process(k1, xin1, ob1)
        out_dma(k1, ob1, so1).start()
        in_dma(k1 + 2, xin1, sx1).start()
        return 0

    lax.fori_loop(0, npr, pair_body, 0)

    K = 2 * npr + 1
    out_dma(K - 1, ob0, so0).wait()
    out_dma(K, ob1, so1).wait()
    in_dma(K + 1, xin0, sx0).wait()
    in_dma(K + 2, xin1, sx1).wait()


def kernel(node_features, batch_indices, size, W1, W2):
    del size
    idx = batch_indices.astype(jnp.int32)
    edges = jnp.arange(0, S + 1, dtype=jnp.int32)
    bounds = jnp.searchsorted(idx, edges, side="left").astype(jnp.int32)
    bounds = jnp.concatenate([bounds, jnp.full((NB - S - 1,), N, jnp.int32)])

    sum_agg, max_agg = _seg_reduce(node_features, bounds)

    refined = pl.pallas_call(
        _mlp_body,
        out_shape=jax.ShapeDtypeStruct((S, C), jnp.float32),
    )(sum_agg, max_agg, W1, W2)

    return _apply_weights(node_features, bounds, refined)
